# baseline (speedup 1.0000x reference)
import jax
import jax.numpy as jnp
from jax import lax
from jax.experimental import pallas as pl
from jax.experimental.pallas import tpu as pltpu

EPS = 1e-5
NB = 4
ADT = jnp.float32
CDT = jnp.float32

_CP = pltpu.CompilerParams(
    dimension_semantics=("parallel",),
    vmem_limit_bytes=32 * 1024 * 1024)


def _taps3(x, L, m0, mL):
    x_m1 = jnp.where(m0, 0.0, pltpu.roll(x, 1, 1))
    x_p1 = jnp.where(mL, 0.0, pltpu.roll(x, L - 1, 1))
    return jnp.concatenate([x_m1, x, x_p1], axis=0)


def _make_stage_kernel(specs, L_out, k3, with_bias, emit_stats, out_dtype):
    def body(*refs):
        idx = 0
        parsed = []
        for sp in specs:
            d = {'mode': sp['mode'], 'data': refs[idx]}
            idx += 1
            if sp['mode'] != 'raw':
                d['scale'], d['shift'] = refs[idx], refs[idx + 1]
                idx += 2
            if sp['mode'] == 'up':
                d['wu'], d['bu'], d['e0'] = refs[idx], refs[idx + 1], refs[idx + 2]
                idx += 3
            parsed.append(d)
        w_ref = refs[idx]; idx += 1
        b_ref = None
        if with_bias:
            b_ref = refs[idx]; idx += 1
        y_ref = refs[idx]; idx += 1
        stats_ref = refs[idx] if emit_stats else None

        w = w_ref[...].astype(CDT)
        bias = b_ref[...] if b_ref is not None else None
        for d in parsed:
            if d['mode'] != 'raw':
                d['sc'] = d['scale'][...]
                d['sh'] = d['shift'][...]
            if d['mode'] == 'up':
                d['wu_v'] = d['wu'][...].astype(CDT)
                d['bu_v'] = d['bu'][...]
                d['es_v'] = d['e0'][...]
        if k3:
            l_idx = lax.broadcasted_iota(jnp.int32, (1, L_out), 1)
            m0, mL = l_idx == 0, l_idx == L_out - 1
        for i in range(NB):
            xs = []
            for d in parsed:
                v = d['data'][i].astype(jnp.float32)
                if d['mode'] != 'raw':
                    v = jnp.maximum(v * d['sc'] + d['sh'], 0.0)
                if d['mode'] == 'up':
                    a = v.astype(CDT)
                    wu, bu = d['wu_v'], d['bu_v']
                    u0 = jnp.dot(wu[0], a, preferred_element_type=jnp.float32) + bu
                    u1 = jnp.dot(wu[1], a, preferred_element_type=jnp.float32) + bu
                    es2 = d['es_v']
                    lp, ts = a.shape[1], es2.shape[0] // 2
                    tiles = []
                    for j in range(lp // ts):
                        sl = slice(j * ts, (j + 1) * ts)
                        pair = jnp.concatenate([u0[:, sl], u1[:, sl]], axis=1)
                        tiles.append(jnp.dot(pair, es2,
                                             preferred_element_type=jnp.float32))
                    v = jnp.concatenate(tiles, axis=1)
                xs.append(v)
            x = xs[0] if len(xs) == 1 else jnp.concatenate(xs, axis=0)
            xk = (_taps3(x, L_out, m0, mL) if k3 else x).astype(CDT)
            y = jnp.dot(w, xk, preferred_element_type=jnp.float32)
            if bias is not None:
                y = y + bias
            y_ref[i] = y.astype(out_dtype)
            if emit_stats:
                s = jnp.sum(y, axis=1, keepdims=True)
                ss = jnp.sum(y * y, axis=1, keepdims=True)
                stats_ref[i] = jnp.concatenate([s, ss], axis=1)

    return body


def _stage(inputs, w_fused, *, k3=True, bias=None, emit_stats=True,
           out_dtype=ADT, L_out=None):
    N = inputs[0]['data'].shape[0]
    G = N // NB
    Cout = w_fused.shape[0]
    if L_out is None:
        L_in = inputs[0]['data'].shape[2]
        L_out = 2 * L_in if inputs[0]['mode'] == 'up' else L_in

    specs, in_specs, args = [], [], []
    for d in inputs:
        C, L = d['data'].shape[1], d['data'].shape[2]
        specs.append({'mode': d['mode']})
        in_specs.append(pl.BlockSpec((NB, C, L), lambda g: (g, 0, 0)))
        args.append(d['data'])
        if d['mode'] != 'raw':
            in_specs += [pl.BlockSpec((C, 1), lambda g: (0, 0)),
                         pl.BlockSpec((C, 1), lambda g: (0, 0))]
            args += [d['scale'], d['shift']]
        if d['mode'] == 'up':
            wu = d['wu']
            in_specs.append(pl.BlockSpec(wu.shape, lambda g: (0, 0, 0)))
            args.append(wu)
            in_specs.append(pl.BlockSpec(d['bu'].shape, lambda g: (0, 0)))
            args.append(d['bu'])
            ts = min(128, L)
            cols = jnp.arange(2 * ts)[None, :]
            e_even = jnp.equal(cols, 2 * jnp.arange(ts)[:, None])
            e_odd = jnp.equal(cols, 2 * jnp.arange(ts)[:, None] + 1)
            e2 = jnp.concatenate([e_even, e_odd], axis=0).astype(jnp.float32)
            in_specs.append(pl.BlockSpec((2 * ts, 2 * ts), lambda g: (0, 0)))
            args.append(e2)
    in_specs.append(pl.BlockSpec(w_fused.shape, lambda g: (0, 0)))
    args.append(w_fused)
    with_bias = bias is not None
    if with_bias:
        in_specs.append(pl.BlockSpec(bias.shape, lambda g: (0, 0)))
        args.append(bias)

    out_specs = [pl.BlockSpec((NB, Cout, L_out), lambda g: (g, 0, 0))]
    out_shape = [jax.ShapeDtypeStruct((N, Cout, L_out), out_dtype)]
    if emit_stats:
        out_specs.append(pl.BlockSpec((NB, Cout, 2), lambda g: (g, 0, 0)))
        out_shape.append(jax.ShapeDtypeStruct((N, Cout, 2), jnp.float32))

    body = _make_stage_kernel(specs, L_out, k3, with_bias, emit_stats, out_dtype)
    outs = pl.pallas_call(
        body,
        grid=(G,),
        in_specs=in_specs,
        out_specs=tuple(out_specs),
        out_shape=tuple(out_shape),
        compiler_params=_CP,
    )(*args)
    return outs if emit_stats else outs[0]


def _scale_shift(stats, gamma, beta, count):
    s = jnp.sum(stats[..., 0], axis=0)
    ss = jnp.sum(stats[..., 1], axis=0)
    mean = s / count
    var = jnp.maximum(ss / count - mean * mean, 0.0)
    inv = lax.rsqrt(var + EPS)
    scale = gamma * inv
    shift = beta - mean * scale
    return scale.reshape(-1, 1), shift.reshape(-1, 1)


def _fuse_w(w):
    cout, cin, _ = w.shape
    return jnp.transpose(w, (0, 2, 1)).reshape(cout, 3 * cin).astype(CDT)


def _block(in1, params, count):
    w1, g1, be1, w2, g2, be2 = params
    y1, st1 = _stage(in1, _fuse_w(w1))
    sc1, sh1 = _scale_shift(st1, g1, be1, count)
    y2, st2 = _stage([{'mode': 'bn', 'data': y1, 'scale': sc1, 'shift': sh1}],
                     _fuse_w(w2))
    sc2, sh2 = _scale_shift(st2, g2, be2, count)
    return y2, sc2, sh2


def kernel(x, enc1_w1, enc1_b1, enc1_g1, enc1_be1, enc1_w2, enc1_b2, enc1_g2, enc1_be2, enc2_w1, enc2_b1, enc2_g1, enc2_be1, enc2_w2, enc2_b2, enc2_g2, enc2_be2, enc3_w1, enc3_b1, enc3_g1, enc3_be1, enc3_w2, enc3_b2, enc3_g2, enc3_be2, enc4_w1, enc4_b1, enc4_g1, enc4_be1, enc4_w2, enc4_b2, enc4_g2, enc4_be2, bottleneck_w1, bottleneck_b1, bottleneck_g1, bottleneck_be1, bottleneck_w2, bottleneck_b2, bottleneck_g2, bottleneck_be2, dec4_w1, dec4_b1, dec4_g1, dec4_be1, dec4_w2, dec4_b2, dec4_g2, dec4_be2, dec3_w1, dec3_b1, dec3_g1, dec3_be1, dec3_w2, dec3_b2, dec3_g2, dec3_be2, dec2_w1, dec2_b1, dec2_g1, dec2_be1, dec2_w2, dec2_b2, dec2_g2, dec2_be2, dec1_w1, dec1_b1, dec1_g1, dec1_be1, dec1_w2, dec1_b2, dec1_g2, dec1_be2, up4_w, up4_b, up3_w, up3_b, up2_w, up2_b, up1_w, up1_b, final_wf, final_bf):
    N, _, L = x.shape
    count = float(N * L)

    def bp(w1, g1, be1, w2, g2, be2):
        return (w1, g1, be1, w2, g2, be2)

    def pool_act(y, sc, sh):
        a = jnp.maximum(y * sc[None, :, :] + sh[None, :, :], 0.0)
        N_, C_, L_ = a.shape
        return jnp.max(a.reshape(N_, C_, L_ // 2, 2), axis=-1)

    e1, sc_e1, sh_e1 = _block([{'mode': 'raw', 'data': x}],
                              bp(enc1_w1, enc1_g1, enc1_be1,
                                 enc1_w2, enc1_g2, enc1_be2), count)
    e2, sc_e2, sh_e2 = _block(
        [{'mode': 'raw', 'data': pool_act(e1, sc_e1, sh_e1)}],
        bp(enc2_w1, enc2_g1, enc2_be1, enc2_w2, enc2_g2, enc2_be2),
        count / 2)
    e3, sc_e3, sh_e3 = _block(
        [{'mode': 'raw', 'data': pool_act(e2, sc_e2, sh_e2)}],
        bp(enc3_w1, enc3_g1, enc3_be1, enc3_w2, enc3_g2, enc3_be2),
        count / 4)
    e4, sc_e4, sh_e4 = _block(
        [{'mode': 'raw', 'data': pool_act(e3, sc_e3, sh_e3)}],
        bp(enc4_w1, enc4_g1, enc4_be1, enc4_w2, enc4_g2, enc4_be2),
        count / 8)
    bt, sc_bt, sh_bt = _block(
        [{'mode': 'raw', 'data': pool_act(e4, sc_e4, sh_e4)}],
        bp(bottleneck_w1, bottleneck_g1, bottleneck_be1,
           bottleneck_w2, bottleneck_g2, bottleneck_be2),
        count / 16)

    def up_in(prev, sc, sh, uw, ub):
        return {'mode': 'up', 'data': prev, 'scale': sc, 'shift': sh,
                'wu': jnp.transpose(uw, (2, 1, 0)).astype(CDT),
                'bu': ub.reshape(-1, 1)}

    def skip_in(e, sc, sh):
        return {'mode': 'bn', 'data': e, 'scale': sc, 'shift': sh}

    d4, sc_d4, sh_d4 = _block(
        [up_in(bt, sc_bt, sh_bt, up4_w, up4_b), skip_in(e4, sc_e4, sh_e4)],
        bp(dec4_w1, dec4_g1, dec4_be1, dec4_w2, dec4_g2, dec4_be2),
        count / 8)
    d3, sc_d3, sh_d3 = _block(
        [up_in(d4, sc_d4, sh_d4, up3_w, up3_b), skip_in(e3, sc_e3, sh_e3)],
        bp(dec3_w1, dec3_g1, dec3_be1, dec3_w2, dec3_g2, dec3_be2),
        count / 4)
    d2, sc_d2, sh_d2 = _block(
        [up_in(d3, sc_d3, sh_d3, up2_w, up2_b), skip_in(e2, sc_e2, sh_e2)],
        bp(dec2_w1, dec2_g1, dec2_be1, dec2_w2, dec2_g2, dec2_be2),
        count / 2)
    d1, sc_d1, sh_d1 = _block(
        [up_in(d2, sc_d2, sh_d2, up1_w, up1_b), skip_in(e1, sc_e1, sh_e1)],
        bp(dec1_w1, dec1_g1, dec1_be1, dec1_w2, dec1_g2, dec1_be2),
        count)

    out = _stage([{'mode': 'bn', 'data': d1, 'scale': sc_d1, 'shift': sh_d1}],
                 final_wf[:, :, 0].astype(CDT), k3=False,
                 bias=final_bf.reshape(-1, 1), emit_stats=False,
                 out_dtype=jnp.float32)
    return out


# speedup vs baseline: 1.1604x; 1.1604x over previous
import jax
import jax.numpy as jnp
from jax import lax
from jax.experimental import pallas as pl
from jax.experimental.pallas import tpu as pltpu

EPS = 1e-5
NB = 8
ADT = jnp.float32
CDT = jnp.float32

_CP = pltpu.CompilerParams(
    dimension_semantics=("parallel",),
    vmem_limit_bytes=32 * 1024 * 1024)


def _taps3(x, L, m0, mL):
    x_m1 = jnp.where(m0, 0.0, pltpu.roll(x, 1, 1))
    x_p1 = jnp.where(mL, 0.0, pltpu.roll(x, L - 1, 1))
    return jnp.concatenate([x_m1, x, x_p1], axis=0)


def _make_stage_kernel(specs, L_out, k3, with_bias, emit_stats, out_dtype):
    def body(*refs):
        idx = 0
        parsed = []
        for sp in specs:
            d = {'mode': sp['mode'], 'data': refs[idx]}
            idx += 1
            if sp['mode'] != 'raw':
                d['scale'], d['shift'] = refs[idx], refs[idx + 1]
                idx += 2
            if sp['mode'] == 'up':
                d['wu'], d['bu'], d['e0'] = refs[idx], refs[idx + 1], refs[idx + 2]
                idx += 3
            parsed.append(d)
        w_ref = refs[idx]; idx += 1
        b_ref = None
        if with_bias:
            b_ref = refs[idx]; idx += 1
        y_ref = refs[idx]; idx += 1
        stats_ref = refs[idx] if emit_stats else None

        w = w_ref[...].astype(CDT)
        bias = b_ref[...] if b_ref is not None else None
        for d in parsed:
            if d['mode'] != 'raw':
                d['sc'] = d['scale'][...]
                d['sh'] = d['shift'][...]
            if d['mode'] == 'up':
                d['wu_v'] = d['wu'][...].astype(CDT)
                d['bu_v'] = d['bu'][...]
                d['es_v'] = d['e0'][...]
        if k3:
            l_idx = lax.broadcasted_iota(jnp.int32, (1, L_out), 1)
            m0, mL = l_idx == 0, l_idx == L_out - 1
        for i in range(NB):
            xs = []
            for d in parsed:
                v = d['data'][i].astype(jnp.float32)
                if d['mode'] != 'raw':
                    v = jnp.maximum(v * d['sc'] + d['sh'], 0.0)
                if d['mode'] == 'up':
                    a = v.astype(CDT)
                    wu, bu = d['wu_v'], d['bu_v']
                    u0 = jnp.dot(wu[0], a, preferred_element_type=jnp.float32) + bu
                    u1 = jnp.dot(wu[1], a, preferred_element_type=jnp.float32) + bu
                    es2 = d['es_v']
                    lp, ts = a.shape[1], es2.shape[0] // 2
                    tiles = []
                    for j in range(lp // ts):
                        sl = slice(j * ts, (j + 1) * ts)
                        pair = jnp.concatenate([u0[:, sl], u1[:, sl]], axis=1)
                        tiles.append(jnp.dot(pair, es2,
                                             preferred_element_type=jnp.float32))
                    v = jnp.concatenate(tiles, axis=1)
                xs.append(v)
            x = xs[0] if len(xs) == 1 else jnp.concatenate(xs, axis=0)
            xk = (_taps3(x, L_out, m0, mL) if k3 else x).astype(CDT)
            y = jnp.dot(w, xk, preferred_element_type=jnp.float32)
            if bias is not None:
                y = y + bias
            y_ref[i] = y.astype(out_dtype)
            if emit_stats:
                s = jnp.sum(y, axis=1, keepdims=True)
                ss = jnp.sum(y * y, axis=1, keepdims=True)
                stats_ref[i] = jnp.concatenate([s, ss], axis=1)

    return body


def _stage(inputs, w_fused, *, k3=True, bias=None, emit_stats=True,
           out_dtype=ADT, L_out=None):
    N = inputs[0]['data'].shape[0]
    G = N // NB
    Cout = w_fused.shape[0]
    if L_out is None:
        L_in = inputs[0]['data'].shape[2]
        L_out = 2 * L_in if inputs[0]['mode'] == 'up' else L_in

    specs, in_specs, args = [], [], []
    for d in inputs:
        C, L = d['data'].shape[1], d['data'].shape[2]
        specs.append({'mode': d['mode']})
        in_specs.append(pl.BlockSpec((NB, C, L), lambda g: (g, 0, 0)))
        args.append(d['data'])
        if d['mode'] != 'raw':
            in_specs += [pl.BlockSpec((C, 1), lambda g: (0, 0)),
                         pl.BlockSpec((C, 1), lambda g: (0, 0))]
            args += [d['scale'], d['shift']]
        if d['mode'] == 'up':
            wu = d['wu']
            in_specs.append(pl.BlockSpec(wu.shape, lambda g: (0, 0, 0)))
            args.append(wu)
            in_specs.append(pl.BlockSpec(d['bu'].shape, lambda g: (0, 0)))
            args.append(d['bu'])
            ts = min(128, L)
            cols = jnp.arange(2 * ts)[None, :]
            e_even = jnp.equal(cols, 2 * jnp.arange(ts)[:, None])
            e_odd = jnp.equal(cols, 2 * jnp.arange(ts)[:, None] + 1)
            e2 = jnp.concatenate([e_even, e_odd], axis=0).astype(jnp.float32)
            in_specs.append(pl.BlockSpec((2 * ts, 2 * ts), lambda g: (0, 0)))
            args.append(e2)
    in_specs.append(pl.BlockSpec(w_fused.shape, lambda g: (0, 0)))
    args.append(w_fused)
    with_bias = bias is not None
    if with_bias:
        in_specs.append(pl.BlockSpec(bias.shape, lambda g: (0, 0)))
        args.append(bias)

    out_specs = [pl.BlockSpec((NB, Cout, L_out), lambda g: (g, 0, 0))]
    out_shape = [jax.ShapeDtypeStruct((N, Cout, L_out), out_dtype)]
    if emit_stats:
        out_specs.append(pl.BlockSpec((NB, Cout, 2), lambda g: (g, 0, 0)))
        out_shape.append(jax.ShapeDtypeStruct((N, Cout, 2), jnp.float32))

    body = _make_stage_kernel(specs, L_out, k3, with_bias, emit_stats, out_dtype)
    outs = pl.pallas_call(
        body,
        grid=(G,),
        in_specs=in_specs,
        out_specs=tuple(out_specs),
        out_shape=tuple(out_shape),
        compiler_params=_CP,
    )(*args)
    return outs if emit_stats else outs[0]


def _scale_shift(stats, gamma, beta, count):
    s = jnp.sum(stats[..., 0], axis=0)
    ss = jnp.sum(stats[..., 1], axis=0)
    mean = s / count
    var = jnp.maximum(ss / count - mean * mean, 0.0)
    inv = lax.rsqrt(var + EPS)
    scale = gamma * inv
    shift = beta - mean * scale
    return scale.reshape(-1, 1), shift.reshape(-1, 1)


def _fuse_w(w):
    cout, cin, _ = w.shape
    return jnp.transpose(w, (0, 2, 1)).reshape(cout, 3 * cin).astype(CDT)


def _block(in1, params, count):
    w1, g1, be1, w2, g2, be2 = params
    y1, st1 = _stage(in1, _fuse_w(w1))
    sc1, sh1 = _scale_shift(st1, g1, be1, count)
    y2, st2 = _stage([{'mode': 'bn', 'data': y1, 'scale': sc1, 'shift': sh1}],
                     _fuse_w(w2))
    sc2, sh2 = _scale_shift(st2, g2, be2, count)
    return y2, sc2, sh2


def kernel(x, enc1_w1, enc1_b1, enc1_g1, enc1_be1, enc1_w2, enc1_b2, enc1_g2, enc1_be2, enc2_w1, enc2_b1, enc2_g1, enc2_be1, enc2_w2, enc2_b2, enc2_g2, enc2_be2, enc3_w1, enc3_b1, enc3_g1, enc3_be1, enc3_w2, enc3_b2, enc3_g2, enc3_be2, enc4_w1, enc4_b1, enc4_g1, enc4_be1, enc4_w2, enc4_b2, enc4_g2, enc4_be2, bottleneck_w1, bottleneck_b1, bottleneck_g1, bottleneck_be1, bottleneck_w2, bottleneck_b2, bottleneck_g2, bottleneck_be2, dec4_w1, dec4_b1, dec4_g1, dec4_be1, dec4_w2, dec4_b2, dec4_g2, dec4_be2, dec3_w1, dec3_b1, dec3_g1, dec3_be1, dec3_w2, dec3_b2, dec3_g2, dec3_be2, dec2_w1, dec2_b1, dec2_g1, dec2_be1, dec2_w2, dec2_b2, dec2_g2, dec2_be2, dec1_w1, dec1_b1, dec1_g1, dec1_be1, dec1_w2, dec1_b2, dec1_g2, dec1_be2, up4_w, up4_b, up3_w, up3_b, up2_w, up2_b, up1_w, up1_b, final_wf, final_bf):
    N, _, L = x.shape
    count = float(N * L)

    def bp(w1, g1, be1, w2, g2, be2):
        return (w1, g1, be1, w2, g2, be2)

    def pool_act(y, sc, sh):
        a = jnp.maximum(y * sc[None, :, :] + sh[None, :, :], 0.0)
        N_, C_, L_ = a.shape
        return jnp.max(a.reshape(N_, C_, L_ // 2, 2), axis=-1)

    e1, sc_e1, sh_e1 = _block([{'mode': 'raw', 'data': x}],
                              bp(enc1_w1, enc1_g1, enc1_be1,
                                 enc1_w2, enc1_g2, enc1_be2), count)
    e2, sc_e2, sh_e2 = _block(
        [{'mode': 'raw', 'data': pool_act(e1, sc_e1, sh_e1)}],
        bp(enc2_w1, enc2_g1, enc2_be1, enc2_w2, enc2_g2, enc2_be2),
        count / 2)
    e3, sc_e3, sh_e3 = _block(
        [{'mode': 'raw', 'data': pool_act(e2, sc_e2, sh_e2)}],
        bp(enc3_w1, enc3_g1, enc3_be1, enc3_w2, enc3_g2, enc3_be2),
        count / 4)
    e4, sc_e4, sh_e4 = _block(
        [{'mode': 'raw', 'data': pool_act(e3, sc_e3, sh_e3)}],
        bp(enc4_w1, enc4_g1, enc4_be1, enc4_w2, enc4_g2, enc4_be2),
        count / 8)
    bt, sc_bt, sh_bt = _block(
        [{'mode': 'raw', 'data': pool_act(e4, sc_e4, sh_e4)}],
        bp(bottleneck_w1, bottleneck_g1, bottleneck_be1,
           bottleneck_w2, bottleneck_g2, bottleneck_be2),
        count / 16)

    def up_in(prev, sc, sh, uw, ub):
        return {'mode': 'up', 'data': prev, 'scale': sc, 'shift': sh,
                'wu': jnp.transpose(uw, (2, 1, 0)).astype(CDT),
                'bu': ub.reshape(-1, 1)}

    def skip_in(e, sc, sh):
        return {'mode': 'bn', 'data': e, 'scale': sc, 'shift': sh}

    d4, sc_d4, sh_d4 = _block(
        [up_in(bt, sc_bt, sh_bt, up4_w, up4_b), skip_in(e4, sc_e4, sh_e4)],
        bp(dec4_w1, dec4_g1, dec4_be1, dec4_w2, dec4_g2, dec4_be2),
        count / 8)
    d3, sc_d3, sh_d3 = _block(
        [up_in(d4, sc_d4, sh_d4, up3_w, up3_b), skip_in(e3, sc_e3, sh_e3)],
        bp(dec3_w1, dec3_g1, dec3_be1, dec3_w2, dec3_g2, dec3_be2),
        count / 4)
    d2, sc_d2, sh_d2 = _block(
        [up_in(d3, sc_d3, sh_d3, up2_w, up2_b), skip_in(e2, sc_e2, sh_e2)],
        bp(dec2_w1, dec2_g1, dec2_be1, dec2_w2, dec2_g2, dec2_be2),
        count / 2)
    d1, sc_d1, sh_d1 = _block(
        [up_in(d2, sc_d2, sh_d2, up1_w, up1_b), skip_in(e1, sc_e1, sh_e1)],
        bp(dec1_w1, dec1_g1, dec1_be1, dec1_w2, dec1_g2, dec1_be2),
        count)

    out = _stage([{'mode': 'bn', 'data': d1, 'scale': sc_d1, 'shift': sh_d1}],
                 final_wf[:, :, 0].astype(CDT), k3=False,
                 bias=final_bf.reshape(-1, 1), emit_stats=False,
                 out_dtype=jnp.float32)
    return out
